# baseline (speedup 1.0000x reference)
import functools
import math

import jax
import jax.numpy as jnp
from jax.experimental import pallas as pl
from jax.experimental.pallas import tpu as pltpu

_K = 2
_NUM_ENC = 5
_TWO_PI = 2.0 * math.pi
_INV_PI = 1.0 / math.pi


def _gmm_body(wi_ref, cond_ref, w1a_ref, w1s_ref, w1c_ref, b1_ref,
              w2t_ref, b2_ref, out_ref, *, freqs):
    cond_t = cond_ref[...].T
    wi_t = wi_ref[...].T
    wx = wi_t[0:1, :]
    wy = wi_t[1:2, :]

    del freqs
    f_col = 2.0 ** jax.lax.broadcasted_iota(
        jnp.int32, (_NUM_ENC, 1), 0).astype(jnp.float32)
    ux = f_col * cond_t[8:9, :]
    uy = f_col * cond_t[9:10, :]
    u = jnp.concatenate([ux, uy], axis=0)

    hpre = (jnp.dot(w1a_ref[...], cond_t, preferred_element_type=jnp.float32)
            + jnp.dot(w1s_ref[...], jnp.sin(u), preferred_element_type=jnp.float32)
            + jnp.dot(w1c_ref[...], jnp.cos(u), preferred_element_type=jnp.float32)
            + b1_ref[...])
    h = jnp.maximum(hpre, 0.0)
    ret = jnp.dot(w2t_ref[...], h, preferred_element_type=jnp.float32) + b2_ref[...]

    pdf = jnp.zeros_like(wx)
    for k in range(_K):
        lsx = ret[4 + 2 * k:5 + 2 * k, :]
        lsy = ret[5 + 2 * k:6 + 2 * k, :]
        zx = (wx - ret[2 * k:2 * k + 1, :]) * jnp.exp(-lsx)
        zy = (wy - ret[2 * k + 1:2 * k + 2, :]) * jnp.exp(-lsy)
        g = jnp.exp(-0.5 * (zx * zx + zy * zy) - lsx - lsy)
        pdf = pdf + jnp.abs(ret[8 + k:9 + k, :]) * g
    pdf = pdf * (1.0 / _TWO_PI)

    w_uni = jnp.abs(ret[10:11, :])
    inside = jnp.where(wx * wx + wy * wy <= 1.0, _INV_PI, 0.0)
    pdf = pdf + w_uni * inside

    wsum = jnp.abs(ret[8:9, :]) + jnp.abs(ret[9:10, :]) + w_uni
    out_ref[...] = (pdf / jnp.maximum(wsum, 1e-12))[None]


@jax.jit
def kernel(wi, cond, W1, b1, W2, b2):
    B = wi.shape[0]
    N = 2048
    G = B // N

    freqs = tuple(2.0 ** i for i in range(_NUM_ENC))

    w1t = W1.T
    cond_cols = [22 + j for j in range(8)] + [0, 1]
    sin_cols = [2 + 4 * i for i in range(_NUM_ENC)] + [3 + 4 * i for i in range(_NUM_ENC)]
    cos_cols = [4 + 4 * i for i in range(_NUM_ENC)] + [5 + 4 * i for i in range(_NUM_ENC)]
    w1a = w1t[:, jnp.asarray(cond_cols)]
    w1s = w1t[:, jnp.asarray(sin_cols)]
    w1c = w1t[:, jnp.asarray(cos_cols)]

    b1c = b1.reshape(-1, 1)
    b2c = b2.reshape(-1, 1)
    w2t = W2.T

    row_spec = lambda c: pl.BlockSpec((N, c), lambda i: (i, 0))
    full = lambda a: pl.BlockSpec(a.shape, lambda i: (0,) * a.ndim)

    pdf = pl.pallas_call(
        functools.partial(_gmm_body, freqs=freqs),
        grid=(G,),
        in_specs=[
            row_spec(2),
            row_spec(10),
            full(w1a), full(w1s), full(w1c), full(b1c),
            full(w2t), full(b2c),
        ],
        out_specs=pl.BlockSpec((1, 1, N), lambda i: (i, 0, 0)),
        out_shape=jax.ShapeDtypeStruct((G, 1, N), jnp.float32),
        compiler_params=pltpu.CompilerParams(
            dimension_semantics=("parallel",)),
    )(wi, cond, w1a, w1s, w1c, b1c, w2t, b2c)

    return (wi, pdf.reshape(B))


# speedup vs baseline: 2.2123x; 2.2123x over previous
import functools
import math

import jax
import jax.numpy as jnp
from jax.experimental import pallas as pl
from jax.experimental.pallas import tpu as pltpu

_K = 2
_NUM_ENC = 5
_TWO_PI = 2.0 * math.pi
_INV_PI = 1.0 / math.pi


def _gmm_body(wi_ref, cond_ref, fsel_ref, w1a_ref, w1s_ref, w1c_ref, b1_ref,
              w2t_ref, b2_ref, out_ref):
    cond_t = cond_ref[...]
    wi_t = wi_ref[...]
    wx = wi_t[0:1, :]
    wy = wi_t[1:2, :]

    u = jnp.dot(fsel_ref[...], cond_t,
                preferred_element_type=jnp.float32)

    hpre = (jnp.dot(w1a_ref[...], cond_t, preferred_element_type=jnp.float32)
            + jnp.dot(w1s_ref[...], jnp.sin(u), preferred_element_type=jnp.float32)
            + jnp.dot(w1c_ref[...], jnp.cos(u), preferred_element_type=jnp.float32)
            + b1_ref[...])
    h = jnp.maximum(hpre, 0.0)
    ret = jnp.dot(w2t_ref[...], h, preferred_element_type=jnp.float32) + b2_ref[...]

    pdf = jnp.zeros_like(wx)
    for k in range(_K):
        lsx = ret[4 + 2 * k:5 + 2 * k, :]
        lsy = ret[5 + 2 * k:6 + 2 * k, :]
        zx = (wx - ret[2 * k:2 * k + 1, :]) * jnp.exp(-lsx)
        zy = (wy - ret[2 * k + 1:2 * k + 2, :]) * jnp.exp(-lsy)
        g = jnp.exp(-0.5 * (zx * zx + zy * zy) - lsx - lsy)
        pdf = pdf + jnp.abs(ret[8 + k:9 + k, :]) * g
    pdf = pdf * (1.0 / _TWO_PI)

    w_uni = jnp.abs(ret[10:11, :])
    inside = jnp.where(wx * wx + wy * wy <= 1.0, _INV_PI, 0.0)
    pdf = pdf + w_uni * inside

    wsum = jnp.abs(ret[8:9, :]) + jnp.abs(ret[9:10, :]) + w_uni
    out_ref[...] = (pdf / jnp.maximum(wsum, 1e-12))[None]


@jax.jit
def kernel(wi, cond, W1, b1, W2, b2):
    B = wi.shape[0]
    N = 2048
    G = B // N

    freqs = jnp.asarray([2.0 ** i for i in range(_NUM_ENC)], jnp.float32)
    fsel = jnp.zeros((2 * _NUM_ENC, 10), jnp.float32)
    fsel = fsel.at[jnp.arange(_NUM_ENC), 8].set(freqs)
    fsel = fsel.at[_NUM_ENC + jnp.arange(_NUM_ENC), 9].set(freqs)

    w1t = W1.T
    cond_cols = [22 + j for j in range(8)] + [0, 1]
    sin_cols = [2 + 4 * i for i in range(_NUM_ENC)] + [3 + 4 * i for i in range(_NUM_ENC)]
    cos_cols = [4 + 4 * i for i in range(_NUM_ENC)] + [5 + 4 * i for i in range(_NUM_ENC)]
    w1a = w1t[:, jnp.asarray(cond_cols)]
    w1s = w1t[:, jnp.asarray(sin_cols)]
    w1c = w1t[:, jnp.asarray(cos_cols)]

    wi_t = wi.T
    cond_t = cond.T
    b1c = b1.reshape(-1, 1)
    b2c = b2.reshape(-1, 1)
    w2t = W2.T

    col_spec = lambda r: pl.BlockSpec((r, N), lambda i: (0, i))
    full = lambda a: pl.BlockSpec(a.shape, lambda i: (0,) * a.ndim)

    pdf = pl.pallas_call(
        _gmm_body,
        grid=(G,),
        in_specs=[
            col_spec(2),
            col_spec(10),
            full(fsel),
            full(w1a), full(w1s), full(w1c), full(b1c),
            full(w2t), full(b2c),
        ],
        out_specs=pl.BlockSpec((1, 1, N), lambda i: (i, 0, 0)),
        out_shape=jax.ShapeDtypeStruct((G, 1, N), jnp.float32),
        compiler_params=pltpu.CompilerParams(
            dimension_semantics=("parallel",)),
    )(wi_t, cond_t, fsel, w1a, w1s, w1c, b1c, w2t, b2c)

    return (wi, pdf.reshape(B))


# speedup vs baseline: 5.5180x; 2.4942x over previous
import math

import jax
import jax.numpy as jnp
from jax.experimental import pallas as pl
from jax.experimental.pallas import tpu as pltpu

_K = 2
_NUM_ENC = 5
_TWO_PI = 2.0 * math.pi
_INV_PI = 1.0 / math.pi

def _sincos(u):
    two_over_pi = 0.63661977236758134
    hi = 1.57080078125
    lo = -4.454454938240815e-06
    lo2 = -1.652011860642233e-13
    del lo2
    k = jnp.round(u * two_over_pi)
    r = u - k * hi
    r = r - k * lo
    ki = k.astype(jnp.int32)

    r2 = r * r
    sp = r * (1.0 + r2 * (-0.16664824 + r2 * 8.1794053e-3))
    cp = 1.0 + r2 * (-0.4997583 + r2 * 4.044261e-2)

    odd = (ki & 1) == 1
    s_base = jnp.where(odd, cp, sp)
    c_base = jnp.where(odd, sp, cp)
    sbit = (ki & 2) << 30
    cbit = ((ki + 1) & 2) << 30
    sin_u = jax.lax.bitcast_convert_type(
        jax.lax.bitcast_convert_type(s_base, jnp.int32) ^ sbit, jnp.float32)
    cos_u = jax.lax.bitcast_convert_type(
        jax.lax.bitcast_convert_type(c_base, jnp.int32) ^ cbit, jnp.float32)
    return sin_u, cos_u


def _gmm_body(wi_ref, cond_ref, w1_ref, b1_ref,
              w2mu_ref, w2ls_ref, w2w_ref, bmu_ref, bls_ref, bw_ref,
              p4_ref, out_ref):
    cond_t = cond_ref[...]
    wi_t = wi_ref[...]

    wi4 = jnp.dot(p4_ref[...], wi_t, preferred_element_type=jnp.float32)
    qwi = wi_t * wi_t
    swi = qwi[0:1, :] + qwi[1:2, :]
    inside = jnp.where(swi <= 1.0, _INV_PI, 0.0)

    f_col = 2.0 ** jax.lax.broadcasted_iota(
        jnp.int32, (_NUM_ENC, 1), 0).astype(jnp.float32)
    ux = f_col * cond_t[8:9, :]
    uy = f_col * cond_t[9:10, :]
    u = jnp.concatenate([ux, uy], axis=0)
    s, c = _sincos(u)

    csc = jnp.concatenate([cond_t, s, c], axis=0)
    hpre = jnp.dot(w1_ref[...], csc,
                   preferred_element_type=jnp.float32) + b1_ref[...]
    h = jnp.maximum(hpre, 0.0)

    mu4 = jnp.dot(w2mu_ref[...], h, preferred_element_type=jnp.float32) + bmu_ref[...]
    ls4 = jnp.dot(w2ls_ref[...], h, preferred_element_type=jnp.float32) + bls_ref[...]
    w3 = jnp.abs(jnp.dot(w2w_ref[...], h, preferred_element_type=jnp.float32)
                 + bw_ref[...])

    einv = jnp.exp(-ls4)
    z = (wi4 - mu4) * einv
    q = z * z
    garg = -0.5 * (q[0:2, :] + q[2:4, :]) - (ls4[0:2, :] + ls4[2:4, :])
    g2 = jnp.exp(garg)

    wg = w3[0:2, :] * g2
    pdfm = wg[0:1, :] + wg[1:2, :]
    w_uni = w3[2:3, :]
    wsum = w3[0:1, :] + w3[1:2, :] + w_uni

    pdf = (pdfm * (1.0 / _TWO_PI) + w_uni * inside)
    pdf = pdf / jnp.maximum(wsum, 1e-12)
    out_ref[...] = pdf[None]


@jax.jit
def kernel(wi, cond, W1, b1, W2, b2):
    B = wi.shape[0]
    N = 16384
    G = B // N

    w1t = W1.T
    cond_cols = [22 + j for j in range(8)] + [0, 1]
    sin_cols = [2 + 4 * i for i in range(_NUM_ENC)] + [3 + 4 * i for i in range(_NUM_ENC)]
    cos_cols = [4 + 4 * i for i in range(_NUM_ENC)] + [5 + 4 * i for i in range(_NUM_ENC)]
    w1cat = w1t[:, jnp.asarray(cond_cols + sin_cols + cos_cols)]

    wi_t = wi.T
    cond_t = cond.T
    b1c = b1.reshape(-1, 1)
    w2t = W2.T
    mu_rows = jnp.asarray([0, 2, 1, 3])
    ls_rows = jnp.asarray([4, 6, 5, 7])
    w2mu, bmu = w2t[mu_rows], b2[mu_rows].reshape(-1, 1)
    w2ls, bls = w2t[ls_rows], b2[ls_rows].reshape(-1, 1)
    w2w, bw = w2t[8:11], b2[8:11].reshape(-1, 1)

    p4 = jnp.asarray([[1., 0.], [1., 0.], [0., 1.], [0., 1.]], jnp.float32)

    col_spec = lambda r: pl.BlockSpec((r, N), lambda i: (0, i))
    full = lambda a: pl.BlockSpec(a.shape, lambda i: (0,) * a.ndim)

    pdf = pl.pallas_call(
        _gmm_body,
        grid=(G,),
        in_specs=[
            col_spec(2),
            col_spec(10),
            full(w1cat), full(b1c),
            full(w2mu), full(w2ls), full(w2w),
            full(bmu), full(bls), full(bw),
            full(p4),
        ],
        out_specs=pl.BlockSpec((1, 1, N), lambda i: (i, 0, 0)),
        out_shape=jax.ShapeDtypeStruct((G, 1, N), jnp.float32),
        compiler_params=pltpu.CompilerParams(
            dimension_semantics=("parallel",)),
    )(wi_t, cond_t, w1cat, b1c, w2mu, w2ls, w2w, bmu, bls, bw, p4)

    return (wi, pdf.reshape(B))


# speedup vs baseline: 5.6585x; 1.0255x over previous
import math

import jax
import jax.numpy as jnp
from jax.experimental import pallas as pl
from jax.experimental.pallas import tpu as pltpu

_K = 2
_NUM_ENC = 5
_TWO_PI = 2.0 * math.pi
_INV_PI = 1.0 / math.pi

def _sincos(u):
    two_over_pi = 0.63661977236758134
    hi = 1.57080078125
    lo = -4.454454938240815e-06
    lo2 = -1.652011860642233e-13
    del lo2
    k = jnp.round(u * two_over_pi)
    r = u - k * hi
    r = r - k * lo
    ki = k.astype(jnp.int32)

    r2 = r * r
    sp = r * (1.0 + r2 * (-0.16664824 + r2 * 8.1794053e-3))
    cp = 1.0 + r2 * (-0.4997583 + r2 * 4.044261e-2)

    odd = (ki & 1) == 1
    s_base = jnp.where(odd, cp, sp)
    c_base = jnp.where(odd, sp, cp)
    sbit = (ki & 2) << 30
    cbit = ((ki + 1) & 2) << 30
    sin_u = jax.lax.bitcast_convert_type(
        jax.lax.bitcast_convert_type(s_base, jnp.int32) ^ sbit, jnp.float32)
    cos_u = jax.lax.bitcast_convert_type(
        jax.lax.bitcast_convert_type(c_base, jnp.int32) ^ cbit, jnp.float32)
    return sin_u, cos_u


def _gmm_body(wi_ref, cond_ref, w1_ref, b1_ref,
              w2mu_ref, w2ls_ref, w2w_ref, bmu_ref, bls_ref, bw_ref,
              p4_ref, out_ref):
    cond_t = cond_ref[...]
    wi_t = wi_ref[...]

    wi4 = jnp.dot(p4_ref[...], wi_t, preferred_element_type=jnp.float32)
    qwi = wi_t * wi_t
    swi = qwi[0:1, :] + qwi[1:2, :]
    inside = jnp.where(swi <= 1.0, _INV_PI, 0.0)

    f_col = 2.0 ** jax.lax.broadcasted_iota(
        jnp.int32, (_NUM_ENC, 1), 0).astype(jnp.float32)
    ux = f_col * cond_t[8:9, :]
    uy = f_col * cond_t[9:10, :]
    u = jnp.concatenate([ux, uy], axis=0)
    s, c = _sincos(u)

    csc = jnp.concatenate([cond_t, s, c], axis=0)
    hpre = jnp.dot(w1_ref[...], csc,
                   preferred_element_type=jnp.float32) + b1_ref[...]
    h = jnp.maximum(hpre, 0.0)

    mu4 = jnp.dot(w2mu_ref[...], h, preferred_element_type=jnp.float32) + bmu_ref[...]
    ls4 = jnp.dot(w2ls_ref[...], h, preferred_element_type=jnp.float32) + bls_ref[...]
    w3 = jnp.abs(jnp.dot(w2w_ref[...], h, preferred_element_type=jnp.float32)
                 + bw_ref[...])

    einv = jnp.exp(-ls4)
    z = (wi4 - mu4) * einv
    q = z * z
    garg = -0.5 * (q[0:2, :] + q[2:4, :]) - (ls4[0:2, :] + ls4[2:4, :])
    g2 = jnp.exp(garg)

    wg = w3[0:2, :] * g2
    pdfm = wg[0:1, :] + wg[1:2, :]
    w_uni = w3[2:3, :]
    wsum = w3[0:1, :] + w3[1:2, :] + w_uni

    pdf = (pdfm * (1.0 / _TWO_PI) + w_uni * inside)
    pdf = pdf / jnp.maximum(wsum, 1e-12)
    out_ref[...] = pdf[None]


@jax.jit
def kernel(wi, cond, W1, b1, W2, b2):
    B = wi.shape[0]
    N = 32768
    G = B // N

    w1t = W1.T
    cond_cols = [22 + j for j in range(8)] + [0, 1]
    sin_cols = [2 + 4 * i for i in range(_NUM_ENC)] + [3 + 4 * i for i in range(_NUM_ENC)]
    cos_cols = [4 + 4 * i for i in range(_NUM_ENC)] + [5 + 4 * i for i in range(_NUM_ENC)]
    w1cat = w1t[:, jnp.asarray(cond_cols + sin_cols + cos_cols)]

    b1c = b1.reshape(-1, 1)
    w2t = W2.T
    mu_rows = jnp.asarray([0, 2, 1, 3])
    ls_rows = jnp.asarray([4, 6, 5, 7])
    w2mu, bmu = w2t[mu_rows], b2[mu_rows].reshape(-1, 1)
    w2ls, bls = w2t[ls_rows], b2[ls_rows].reshape(-1, 1)
    w2w, bw = w2t[8:11], b2[8:11].reshape(-1, 1)

    p4 = jnp.asarray([[1., 0.], [1., 0.], [0., 1.], [0., 1.]], jnp.float32)

    wi_t = wi.T
    cond_t = cond.T

    col_spec = lambda r: pl.BlockSpec((r, N), lambda i: (0, i))
    full = lambda a: pl.BlockSpec(a.shape, lambda i: (0,) * a.ndim)

    pdf = pl.pallas_call(
        _gmm_body,
        grid=(G,),
        in_specs=[
            col_spec(2),
            col_spec(10),
            full(w1cat), full(b1c),
            full(w2mu), full(w2ls), full(w2w),
            full(bmu), full(bls), full(bw),
            full(p4),
        ],
        out_specs=pl.BlockSpec((1, 1, N), lambda i: (i, 0, 0)),
        out_shape=jax.ShapeDtypeStruct((G, 1, N), jnp.float32),
        compiler_params=pltpu.CompilerParams(
            dimension_semantics=("parallel",)),
    )(wi_t, cond_t, w1cat, b1c, w2mu, w2ls, w2w, bmu, bls, bw, p4)

    return (wi, pdf.reshape(B))


# speedup vs baseline: 6.1596x; 1.0886x over previous
import math

import jax
import jax.numpy as jnp
from jax.experimental import pallas as pl
from jax.experimental.pallas import tpu as pltpu

_K = 2
_NUM_ENC = 5
_TWO_PI = 2.0 * math.pi
_INV_PI = 1.0 / math.pi

def _sincos(u):
    two_over_pi = 0.63661977236758134
    hi = 1.57080078125
    lo = -4.454454938240815e-06
    lo2 = -1.652011860642233e-13
    del lo2
    k = jnp.round(u * two_over_pi)
    r = u - k * hi
    r = r - k * lo
    ki = k.astype(jnp.int32)

    r2 = r * r
    sp = r * (1.0 + r2 * (-0.16664824 + r2 * 8.1794053e-3))
    cp = 1.0 + r2 * (-0.4997583 + r2 * 4.044261e-2)

    odd = (ki & 1) == 1
    s_base = jnp.where(odd, cp, sp)
    c_base = jnp.where(odd, sp, cp)
    sgn_s = 1.0 - (ki & 2).astype(jnp.float32)
    sgn_c = 1.0 - ((ki + 1) & 2).astype(jnp.float32)
    return s_base * sgn_s, c_base * sgn_c


def _gmm_body(wi_ref, cond_ref, w1_ref, b1_ref,
              w2mu_ref, w2ls_ref, w2w_ref, bmu_ref, bls_ref, bw_ref,
              p4_ref, out_ref):
    cond_t = cond_ref[...]
    wi_t = wi_ref[...]

    wi4 = jnp.dot(p4_ref[...], wi_t, preferred_element_type=jnp.float32)
    qwi = wi_t * wi_t
    swi = qwi[0:1, :] + qwi[1:2, :]
    inside = jnp.where(swi <= 1.0, _INV_PI, 0.0)

    xy = cond_t[8:10, :]
    fpat = jax.lax.bitcast_convert_type(
        ((jax.lax.broadcasted_iota(jnp.int32, (8, 1), 0) >> 1) + 127) << 23,
        jnp.float32)
    u8 = jnp.concatenate([xy, xy, xy, xy], axis=0) * fpat
    s8, c8 = _sincos(u8)
    s_f3 = s8[6:8, :]
    c_f3 = c8[6:8, :]
    s2 = (s_f3 + s_f3) * c_f3
    c2 = 1.0 - (s_f3 + s_f3) * s_f3

    csc = jnp.concatenate([cond_t, s8, s2, c8, c2], axis=0)
    hpre = jnp.dot(w1_ref[...], csc,
                   preferred_element_type=jnp.float32) + b1_ref[...]
    h = jnp.maximum(hpre, 0.0)

    mu4 = jnp.dot(w2mu_ref[...], h, preferred_element_type=jnp.float32) + bmu_ref[...]
    ls4 = jnp.dot(w2ls_ref[...], h, preferred_element_type=jnp.float32) + bls_ref[...]
    w3 = jnp.abs(jnp.dot(w2w_ref[...], h, preferred_element_type=jnp.float32)
                 + bw_ref[...])

    einv = jnp.exp(-ls4)
    z = (wi4 - mu4) * einv
    q = z * z
    garg = -0.5 * (q[0:2, :] + q[2:4, :]) - (ls4[0:2, :] + ls4[2:4, :])
    g2 = jnp.exp(garg)

    wg = w3[0:2, :] * g2
    pdfm = wg[0:1, :] + wg[1:2, :]
    w_uni = w3[2:3, :]
    wsum = w3[0:1, :] + w3[1:2, :] + w_uni

    pdf = (pdfm * (1.0 / _TWO_PI) + w_uni * inside)
    pdf = pdf / jnp.maximum(wsum, 1e-12)
    out_ref[...] = pdf[None]


@jax.jit
def kernel(wi, cond, W1, b1, W2, b2):
    B = wi.shape[0]
    N = 32768
    G = B // N

    w1t = W1.T
    cond_cols = [22 + j for j in range(8)] + [0, 1]
    sin_cols = [2 + 4 * i + d for i in range(4) for d in (0, 1)] + [18, 19]
    cos_cols = [4 + 4 * i + d for i in range(4) for d in (0, 1)] + [20, 21]
    w1cat = w1t[:, jnp.asarray(cond_cols + sin_cols + cos_cols)]

    b1c = b1.reshape(-1, 1)
    w2t = W2.T
    mu_rows = jnp.asarray([0, 2, 1, 3])
    ls_rows = jnp.asarray([4, 6, 5, 7])
    w2mu, bmu = w2t[mu_rows], b2[mu_rows].reshape(-1, 1)
    w2ls, bls = w2t[ls_rows], b2[ls_rows].reshape(-1, 1)
    w2w, bw = w2t[8:11], b2[8:11].reshape(-1, 1)

    p4 = jnp.asarray([[1., 0.], [1., 0.], [0., 1.], [0., 1.]], jnp.float32)

    wi_t = wi.T
    cond_t = cond.T

    col_spec = lambda r: pl.BlockSpec((r, N), lambda i: (0, i))
    full = lambda a: pl.BlockSpec(a.shape, lambda i: (0,) * a.ndim)

    pdf = pl.pallas_call(
        _gmm_body,
        grid=(G,),
        in_specs=[
            col_spec(2),
            col_spec(10),
            full(w1cat), full(b1c),
            full(w2mu), full(w2ls), full(w2w),
            full(bmu), full(bls), full(bw),
            full(p4),
        ],
        out_specs=pl.BlockSpec((1, 1, N), lambda i: (i, 0, 0)),
        out_shape=jax.ShapeDtypeStruct((G, 1, N), jnp.float32),
        compiler_params=pltpu.CompilerParams(
            dimension_semantics=("parallel",)),
    )(wi_t, cond_t, w1cat, b1c, w2mu, w2ls, w2w, bmu, bls, bw, p4)

    return (wi, pdf.reshape(B))
